# baseline (speedup 1.0000x reference)
import jax
import jax.numpy as jnp
from jax.experimental import pallas as pl
from jax.experimental.pallas import tpu as pltpu

_BM = 8192
_CHUNK = 2048

_DN_RHS_T = (((1,), (1,)), ((), ()))


def _nfm_body(x_ref, wlin_ref, bw_ref, V_ref, w1_ref, b1_ref, w2_ref, b2_ref,
              w3_ref, b3_ref, wout_ref, bo_ref, o_ref):
    V = V_ref[...]
    V2 = V * V
    w1 = w1_ref[...]
    w2 = w2_ref[...]
    w3 = w3_ref[...]
    wlin_t = wlin_ref[...]
    wout_h = wout_ref[...] * 0.5
    b1 = b1_ref[...].reshape(1, -1) * 2.0
    b2 = b2_ref[...].reshape(1, -1) * 2.0
    b3 = b3_ref[...].reshape(1, -1) * 2.0
    bias = (bw_ref[...] + bo_ref[...]).reshape(1, 1)
    for c in range(_BM // _CHUNK):
        rows = pl.ds(c * _CHUNK, _CHUNK)
        x = x_ref[rows, :]
        xv = jnp.dot(x, V, preferred_element_type=jnp.float32)
        x2v2 = jnp.dot(x * x, V2, preferred_element_type=jnp.float32)
        t = xv * xv + x2v2
        t = jnp.maximum(jnp.dot(t, w1, preferred_element_type=jnp.float32) + b1, 0.0)
        t = jnp.maximum(jnp.dot(t, w2, preferred_element_type=jnp.float32) + b2, 0.0)
        t = jnp.maximum(jnp.dot(t, w3, preferred_element_type=jnp.float32) + b3, 0.0)
        z = (jax.lax.dot_general(wlin_t, x, _DN_RHS_T, preferred_element_type=jnp.float32)
             + jax.lax.dot_general(wout_h, t, _DN_RHS_T, preferred_element_type=jnp.float32)
             + bias)
        o_ref[:, pl.ds(c * _CHUNK, _CHUNK)] = jax.nn.sigmoid(z)


def kernel(x, w_wide, b_wide, V, w1, b1, w2, b2, w3, b3, w_out, b_out):
    B, F = x.shape
    K = V.shape[1]
    H1 = w1.shape[1]
    H2 = w2.shape[1]
    H3 = w3.shape[1]

    wlin_t = w_wide.reshape(1, F)
    wout_t = w_out.reshape(1, H3)

    grid = (B // _BM,)
    full = lambda shape: pl.BlockSpec(shape, lambda i: (0,) * len(shape))
    out = pl.pallas_call(
        _nfm_body,
        out_shape=jax.ShapeDtypeStruct((1, B), jnp.float32),
        grid=grid,
        in_specs=[
            pl.BlockSpec((_BM, F), lambda i: (i, 0)),
            full((1, F)),
            full((1,)),
            full((F, K)),
            full((K, H1)),
            full((H1,)),
            full((H1, H2)),
            full((H2,)),
            full((H2, H3)),
            full((H3,)),
            full((1, H3)),
            full((1,)),
        ],
        out_specs=pl.BlockSpec((1, _BM), lambda i: (0, i)),
        compiler_params=pltpu.CompilerParams(
            dimension_semantics=("parallel",),
        ),
        name="nfm_fused",
    )(x, wlin_t, b_wide, V, w1, b1, w2, b2, w3, b3, wout_t, b_out)
    return out.reshape(B, 1)


# speedup vs baseline: 1.1002x; 1.1002x over previous
import jax
import jax.numpy as jnp
from jax.experimental import pallas as pl
from jax.experimental.pallas import tpu as pltpu

_BM = 8192
_CHUNK = 8192

_DN_RHS_T = (((1,), (1,)), ((), ()))


def _nfm_body(x_ref, wlin_ref, bw_ref, V_ref, w1_ref, b1_ref, w2_ref, b2_ref,
              w3_ref, b3_ref, wout_ref, bo_ref, o_ref):
    V = V_ref[...]
    V2 = V * V
    w1 = w1_ref[...]
    w2 = w2_ref[...]
    w3 = w3_ref[...]
    wlin_t = wlin_ref[...]
    wout_h = wout_ref[...] * 0.5
    b1 = b1_ref[...].reshape(1, -1) * 2.0
    b2 = b2_ref[...].reshape(1, -1) * 2.0
    b3 = b3_ref[...].reshape(1, -1) * 2.0
    bias = (bw_ref[...] + bo_ref[...]).reshape(1, 1)
    for c in range(_BM // _CHUNK):
        rows = pl.ds(c * _CHUNK, _CHUNK)
        x = x_ref[rows, :]
        xv = jnp.dot(x, V, preferred_element_type=jnp.float32)
        x2v2 = jnp.dot(x * x, V2, preferred_element_type=jnp.float32)
        t = xv * xv + x2v2
        t = jnp.maximum(jnp.dot(t, w1, preferred_element_type=jnp.float32) + b1, 0.0)
        t = jnp.maximum(jnp.dot(t, w2, preferred_element_type=jnp.float32) + b2, 0.0)
        t = jnp.maximum(jnp.dot(t, w3, preferred_element_type=jnp.float32) + b3, 0.0)
        z = (jax.lax.dot_general(wlin_t, x, _DN_RHS_T, preferred_element_type=jnp.float32)
             + jax.lax.dot_general(wout_h, t, _DN_RHS_T, preferred_element_type=jnp.float32)
             + bias)
        o_ref[:, pl.ds(c * _CHUNK, _CHUNK)] = jax.nn.sigmoid(z)


def kernel(x, w_wide, b_wide, V, w1, b1, w2, b2, w3, b3, w_out, b_out):
    B, F = x.shape
    K = V.shape[1]
    H1 = w1.shape[1]
    H2 = w2.shape[1]
    H3 = w3.shape[1]

    wlin_t = w_wide.reshape(1, F)
    wout_t = w_out.reshape(1, H3)

    grid = (B // _BM,)
    full = lambda shape: pl.BlockSpec(shape, lambda i: (0,) * len(shape))
    out = pl.pallas_call(
        _nfm_body,
        out_shape=jax.ShapeDtypeStruct((1, B), jnp.float32),
        grid=grid,
        in_specs=[
            pl.BlockSpec((_BM, F), lambda i: (i, 0)),
            full((1, F)),
            full((1,)),
            full((F, K)),
            full((K, H1)),
            full((H1,)),
            full((H1, H2)),
            full((H2,)),
            full((H2, H3)),
            full((H3,)),
            full((1, H3)),
            full((1,)),
        ],
        out_specs=pl.BlockSpec((1, _BM), lambda i: (0, i)),
        compiler_params=pltpu.CompilerParams(
            dimension_semantics=("parallel",),
        ),
        name="nfm_fused",
    )(x, wlin_t, b_wide, V, w1, b1, w2, b2, w3, b3, wout_t, b_out)
    return out.reshape(B, 1)


# speedup vs baseline: 1.3587x; 1.2349x over previous
import jax
import jax.numpy as jnp
from jax.experimental import pallas as pl
from jax.experimental.pallas import tpu as pltpu

_BM = 8192
_CHUNK = 4096

_DN_RHS_T = (((1,), (1,)), ((), ()))
_DN_LHS_T = (((0,), (1,)), ((), ()))
_DN_BOTH0 = (((0,), (0,)), ((), ()))
_DN_LHS_LHS = (((1,), (0,)), ((), ()))


def _nfm_body(x_ref, wlin_ref, bw_ref, V_ref, w1_ref, b1_ref, w2_ref, b2_ref,
              w3_ref, b3_ref, wout_ref, bo_ref, o_ref):
    V = V_ref[...]
    V2 = V * V
    w1 = w1_ref[...]
    w2 = w2_ref[...]
    w3 = w3_ref[...]
    wlin_t = wlin_ref[...]
    wout_h = wout_ref[...] * 0.5
    b1c = b1_ref[...].reshape(-1, 1) * 2.0
    b2c = b2_ref[...].reshape(-1, 1) * 2.0
    b3c = b3_ref[...].reshape(-1, 1) * 2.0
    bias = (bw_ref[...] + bo_ref[...]).reshape(1, 1)
    for c in range(_BM // _CHUNK):
        rows = pl.ds(c * _CHUNK, _CHUNK)
        x = x_ref[rows, :]
        zlin = jax.lax.dot_general(wlin_t, x, _DN_RHS_T, preferred_element_type=jnp.float32)
        xv = jnp.dot(x, V, preferred_element_type=jnp.float32)
        x2v2 = jnp.dot(x * x, V2, preferred_element_type=jnp.float32)
        t = xv * xv + x2v2
        u = jnp.maximum(jax.lax.dot_general(w1, t, _DN_LHS_T, preferred_element_type=jnp.float32) + b1c, 0.0)
        u = jnp.maximum(jax.lax.dot_general(w2, u, _DN_BOTH0, preferred_element_type=jnp.float32) + b2c, 0.0)
        u = jnp.maximum(jax.lax.dot_general(w3, u, _DN_BOTH0, preferred_element_type=jnp.float32) + b3c, 0.0)
        z = (zlin
             + jax.lax.dot_general(wout_h, u, _DN_LHS_LHS, preferred_element_type=jnp.float32)
             + bias)
        o_ref[:, pl.ds(c * _CHUNK, _CHUNK)] = jax.nn.sigmoid(z)


def kernel(x, w_wide, b_wide, V, w1, b1, w2, b2, w3, b3, w_out, b_out):
    B, F = x.shape
    K = V.shape[1]
    H1 = w1.shape[1]
    H2 = w2.shape[1]
    H3 = w3.shape[1]

    wlin_t = w_wide.reshape(1, F)
    wout_t = w_out.reshape(1, H3)

    grid = (B // _BM,)
    full = lambda shape: pl.BlockSpec(shape, lambda i: (0,) * len(shape))
    out = pl.pallas_call(
        _nfm_body,
        out_shape=jax.ShapeDtypeStruct((1, B), jnp.float32),
        grid=grid,
        in_specs=[
            pl.BlockSpec((_BM, F), lambda i: (i, 0)),
            full((1, F)),
            full((1,)),
            full((F, K)),
            full((K, H1)),
            full((H1,)),
            full((H1, H2)),
            full((H2,)),
            full((H2, H3)),
            full((H3,)),
            full((1, H3)),
            full((1,)),
        ],
        out_specs=pl.BlockSpec((1, _BM), lambda i: (0, i)),
        compiler_params=pltpu.CompilerParams(
            dimension_semantics=("parallel",),
        ),
        name="nfm_fused",
    )(x, wlin_t, b_wide, V, w1, b1, w2, b2, w3, b3, wout_t, b_out)
    return out.reshape(B, 1)


# speedup vs baseline: 1.3940x; 1.0260x over previous
import jax
import jax.numpy as jnp
from jax.experimental import pallas as pl
from jax.experimental.pallas import tpu as pltpu

_BM = 8192
_CHUNK = 8192

_DN_RHS_T = (((1,), (1,)), ((), ()))
_DN_LHS_T = (((0,), (1,)), ((), ()))
_DN_BOTH0 = (((0,), (0,)), ((), ()))
_DN_LHS_LHS = (((1,), (0,)), ((), ()))


def _nfm_body(x_ref, wlin_ref, bw_ref, V_ref, w1_ref, b1_ref, w2_ref, b2_ref,
              w3_ref, b3_ref, wout_ref, bo_ref, o_ref):
    V = V_ref[...]
    V2 = V * V
    w1 = w1_ref[...]
    w2 = w2_ref[...]
    w3 = w3_ref[...]
    wlin_t = wlin_ref[...]
    wout_h = wout_ref[...] * 0.5
    b1c = b1_ref[...].reshape(-1, 1) * 2.0
    b2c = b2_ref[...].reshape(-1, 1) * 2.0
    b3c = b3_ref[...].reshape(-1, 1) * 2.0
    bias = (bw_ref[...] + bo_ref[...]).reshape(1, 1)
    for c in range(_BM // _CHUNK):
        rows = pl.ds(c * _CHUNK, _CHUNK)
        x = x_ref[rows, :]
        zlin = jax.lax.dot_general(wlin_t, x, _DN_RHS_T, preferred_element_type=jnp.float32)
        xv = jnp.dot(x, V, preferred_element_type=jnp.float32)
        x2v2 = jnp.dot(x * x, V2, preferred_element_type=jnp.float32)
        t = xv * xv + x2v2
        u = jnp.maximum(jax.lax.dot_general(w1, t, _DN_LHS_T, preferred_element_type=jnp.float32) + b1c, 0.0)
        u = jnp.maximum(jax.lax.dot_general(w2, u, _DN_BOTH0, preferred_element_type=jnp.float32) + b2c, 0.0)
        u = jnp.maximum(jax.lax.dot_general(w3, u, _DN_BOTH0, preferred_element_type=jnp.float32) + b3c, 0.0)
        z = (zlin
             + jax.lax.dot_general(wout_h, u, _DN_LHS_LHS, preferred_element_type=jnp.float32)
             + bias)
        o_ref[:, pl.ds(c * _CHUNK, _CHUNK)] = jax.nn.sigmoid(z)


def kernel(x, w_wide, b_wide, V, w1, b1, w2, b2, w3, b3, w_out, b_out):
    B, F = x.shape
    K = V.shape[1]
    H1 = w1.shape[1]
    H2 = w2.shape[1]
    H3 = w3.shape[1]

    wlin_t = w_wide.reshape(1, F)
    wout_t = w_out.reshape(1, H3)

    grid = (B // _BM,)
    full = lambda shape: pl.BlockSpec(shape, lambda i: (0,) * len(shape))
    out = pl.pallas_call(
        _nfm_body,
        out_shape=jax.ShapeDtypeStruct((1, B), jnp.float32),
        grid=grid,
        in_specs=[
            pl.BlockSpec((_BM, F), lambda i: (i, 0)),
            full((1, F)),
            full((1,)),
            full((F, K)),
            full((K, H1)),
            full((H1,)),
            full((H1, H2)),
            full((H2,)),
            full((H2, H3)),
            full((H3,)),
            full((1, H3)),
            full((1,)),
        ],
        out_specs=pl.BlockSpec((1, _BM), lambda i: (0, i)),
        compiler_params=pltpu.CompilerParams(
            dimension_semantics=("parallel",),
        ),
        name="nfm_fused",
    )(x, wlin_t, b_wide, V, w1, b1, w2, b2, w3, b3, wout_t, b_out)
    return out.reshape(B, 1)
